# baseline (speedup 1.0000x reference)
import functools

import jax
import jax.numpy as jnp
from jax import lax
from jax.experimental import pallas as pl
from jax.experimental.pallas import tpu as pltpu
from jax.experimental.pallas import tpu_sc as plsc

_CHUNK = 128


@functools.lru_cache(maxsize=None)
def _make_gather(vocab, dim, batch):
    info = plsc.get_sparse_core_info()
    num_workers = info.num_cores * info.num_subcores
    b_per_w = batch // num_workers
    n_chunks = b_per_w // _CHUNK
    assert b_per_w % _CHUNK == 0 and batch % num_workers == 0

    mesh = plsc.VectorSubcoreMesh(core_axis_name="c", subcore_axis_name="s")

    @functools.partial(
        pl.kernel,
        mesh=mesh,
        out_type=jax.ShapeDtypeStruct((batch, dim), jnp.float32),
        scratch_types=[
            pltpu.VMEM((n_chunks, _CHUNK), jnp.int32),
            pltpu.VMEM((b_per_w, dim), jnp.float32),
            pltpu.SemaphoreType.DMA((n_chunks,)),
            pltpu.SemaphoreType.DMA,
        ],
    )
    def gather_kernel(idx_hbm, table_hbm, out_hbm, idx_v, rows_v, gsem, wsem):
        wid = lax.axis_index("s") * info.num_cores + lax.axis_index("c")
        base = wid * b_per_w
        pltpu.sync_copy(idx_hbm.at[wid], idx_v)
        gathers = [
            pltpu.async_copy(
                table_hbm.at[idx_v.at[j]],
                rows_v.at[pl.ds(j * _CHUNK, _CHUNK)],
                gsem.at[j],
            )
            for j in range(n_chunks)
        ]
        writes = []
        for j in range(n_chunks):
            gathers[j].wait()
            writes.append(
                pltpu.async_copy(
                    rows_v.at[pl.ds(j * _CHUNK, _CHUNK)],
                    out_hbm.at[pl.ds(base + j * _CHUNK, _CHUNK)],
                    wsem,
                )
            )
        for w in writes:
            w.wait()

    return gather_kernel


def kernel(timesteps, table):
    batch = timesteps.shape[0]
    vocab, dim = table.shape
    info = plsc.get_sparse_core_info()
    num_workers = info.num_cores * info.num_subcores
    idx = jnp.reshape(
        timesteps.astype(jnp.int32),
        (num_workers, batch // (num_workers * _CHUNK), _CHUNK),
    )
    return _make_gather(vocab, dim, batch)(idx, table)


# speedup vs baseline: 1.1636x; 1.1636x over previous
import functools

import jax
import jax.numpy as jnp
from jax import lax
from jax.experimental import pallas as pl
from jax.experimental.pallas import tpu as pltpu
from jax.experimental.pallas import tpu_sc as plsc

_CHUNK = 128


@functools.lru_cache(maxsize=None)
def _make_gather(vocab, dim, batch):
    info = plsc.get_sparse_core_info()
    num_sub = info.num_subcores
    num_workers = info.num_cores * num_sub
    b_per_w = batch // num_workers
    n_chunks = b_per_w // _CHUNK
    assert b_per_w % _CHUNK == 0 and batch % num_workers == 0
    v_share = (-(-vocab // num_sub) + 7) // 8 * 8
    n_copiers = vocab // v_share
    v_rem = vocab - n_copiers * v_share
    assert v_rem % 8 == 0 and n_copiers + (1 if v_rem else 0) <= num_sub

    mesh = plsc.VectorSubcoreMesh(core_axis_name="c", subcore_axis_name="s")

    @functools.partial(
        pl.kernel,
        mesh=mesh,
        out_type=jax.ShapeDtypeStruct((batch, dim), jnp.float32),
        scratch_types=[
            pltpu.VMEM((n_chunks, _CHUNK), jnp.int32),
            pltpu.VMEM((b_per_w, dim), jnp.float32),
            pltpu.VMEM_SHARED((vocab, dim), jnp.float32),
            pltpu.SemaphoreType.DMA((n_chunks,)),
            pltpu.SemaphoreType.DMA,
        ],
    )
    def gather_kernel(idx_hbm, table_hbm, out_hbm, idx_v, rows_v, table_sp,
                      gsem, wsem):
        sid = lax.axis_index("s")
        wid = sid * info.num_cores + lax.axis_index("c")
        base = wid * b_per_w
        pltpu.sync_copy(idx_hbm.at[wid], idx_v)
        @pl.when(sid < n_copiers)
        def _copy_share():
            row0 = sid * v_share
            pltpu.sync_copy(table_hbm.at[pl.ds(row0, v_share)],
                            table_sp.at[pl.ds(row0, v_share)])

        if v_rem:
            @pl.when(sid == n_copiers)
            def _copy_rem():
                pltpu.sync_copy(
                    table_hbm.at[pl.ds(n_copiers * v_share, v_rem)],
                    table_sp.at[pl.ds(n_copiers * v_share, v_rem)])

        plsc.subcore_barrier()
        gathers = [
            pltpu.async_copy(
                table_sp.at[idx_v.at[j]],
                rows_v.at[pl.ds(j * _CHUNK, _CHUNK)],
                gsem.at[j],
            )
            for j in range(n_chunks)
        ]
        writes = []
        for j in range(n_chunks):
            gathers[j].wait()
            writes.append(
                pltpu.async_copy(
                    rows_v.at[pl.ds(j * _CHUNK, _CHUNK)],
                    out_hbm.at[pl.ds(base + j * _CHUNK, _CHUNK)],
                    wsem,
                )
            )
        for w in writes:
            w.wait()

    return gather_kernel


def kernel(timesteps, table):
    batch = timesteps.shape[0]
    vocab, dim = table.shape
    info = plsc.get_sparse_core_info()
    num_workers = info.num_cores * info.num_subcores
    idx = jnp.reshape(
        timesteps.astype(jnp.int32),
        (num_workers, batch // (num_workers * _CHUNK), _CHUNK),
    )
    return _make_gather(vocab, dim, batch)(idx, table)


# speedup vs baseline: 1.1708x; 1.0062x over previous
import functools

import jax
import jax.numpy as jnp
from jax import lax
from jax.experimental import pallas as pl
from jax.experimental.pallas import tpu as pltpu
from jax.experimental.pallas import tpu_sc as plsc

_CHUNK = 64


@functools.lru_cache(maxsize=None)
def _make_gather(vocab, dim, batch):
    info = plsc.get_sparse_core_info()
    num_sub = info.num_subcores
    num_workers = info.num_cores * num_sub
    b_per_w = batch // num_workers
    n_chunks = b_per_w // _CHUNK
    assert b_per_w % _CHUNK == 0 and batch % num_workers == 0
    v_share = (-(-vocab // num_sub) + 7) // 8 * 8
    n_copiers = vocab // v_share
    v_rem = vocab - n_copiers * v_share
    assert v_rem % 8 == 0 and n_copiers + (1 if v_rem else 0) <= num_sub

    mesh = plsc.VectorSubcoreMesh(core_axis_name="c", subcore_axis_name="s")

    @functools.partial(
        pl.kernel,
        mesh=mesh,
        out_type=jax.ShapeDtypeStruct((batch, dim), jnp.float32),
        scratch_types=[
            pltpu.VMEM((n_chunks, _CHUNK), jnp.int32),
            pltpu.VMEM((b_per_w, dim), jnp.float32),
            pltpu.VMEM_SHARED((vocab, dim), jnp.float32),
            pltpu.SemaphoreType.DMA((n_chunks,)),
            pltpu.SemaphoreType.DMA,
        ],
    )
    def gather_kernel(idx_hbm, table_hbm, out_hbm, idx_v, rows_v, table_sp,
                      gsem, wsem):
        sid = lax.axis_index("s")
        wid = sid * info.num_cores + lax.axis_index("c")
        base = wid * b_per_w
        @pl.when(sid < n_copiers)
        def _copy_share():
            row0 = sid * v_share
            pltpu.sync_copy(table_hbm.at[pl.ds(row0, v_share)],
                            table_sp.at[pl.ds(row0, v_share)])

        if v_rem:
            @pl.when(sid == n_copiers)
            def _copy_rem():
                pltpu.sync_copy(
                    table_hbm.at[pl.ds(n_copiers * v_share, v_rem)],
                    table_sp.at[pl.ds(n_copiers * v_share, v_rem)])

        pltpu.sync_copy(idx_hbm.at[wid], idx_v)
        plsc.subcore_barrier()
        gathers = [
            pltpu.async_copy(
                table_sp.at[idx_v.at[j]],
                rows_v.at[pl.ds(j * _CHUNK, _CHUNK)],
                gsem.at[j],
            )
            for j in range(n_chunks)
        ]
        writes = []
        for j in range(n_chunks):
            gathers[j].wait()
            writes.append(
                pltpu.async_copy(
                    rows_v.at[pl.ds(j * _CHUNK, _CHUNK)],
                    out_hbm.at[pl.ds(base + j * _CHUNK, _CHUNK)],
                    wsem,
                )
            )
        for w in writes:
            w.wait()

    return gather_kernel


def kernel(timesteps, table):
    batch = timesteps.shape[0]
    vocab, dim = table.shape
    info = plsc.get_sparse_core_info()
    num_workers = info.num_cores * info.num_subcores
    idx = jnp.reshape(
        timesteps.astype(jnp.int32),
        (num_workers, batch // (num_workers * _CHUNK), _CHUNK),
    )
    return _make_gather(vocab, dim, batch)(idx, table)
